# baseline (speedup 1.0000x reference)
import functools

import numpy as np

import jax
import jax.numpy as jnp
from jax import lax
from jax.experimental import pallas as pl
from jax.experimental.pallas import tpu as pltpu
from jax.experimental.pallas import tpu_sc as plsc

B = 4
S = 2048
D = 768
LANES = 16
D_VECS = D // LANES

NUM_CORES = 2
NUM_SUBCORES = 16
SCHUNK = 8
ROWS = SCHUNK * B
GSTEPS = (B * S) // ROWS
SPP = GSTEPS // NUM_SUBCORES
N0 = 11
N1 = SPP - N0
NBUF = 3


def _make_kernel():
    mesh = plsc.VectorSubcoreMesh(core_axis_name="c", subcore_axis_name="s")

    @functools.partial(
        pl.kernel,
        mesh=mesh,
        out_type=jax.ShapeDtypeStruct((B * S, D), jnp.float32),
        scratch_types=[
            pltpu.VMEM((N0 * ROWS,), jnp.int32),
            pltpu.VMEM((N0, ROWS), jnp.int32),
            pltpu.VMEM((NBUF, SCHUNK, D), jnp.float32),
            pltpu.VMEM((ROWS, D), jnp.float32),
            pltpu.VMEM((ROWS, D), jnp.float32),
            pltpu.VMEM((ROWS, D), jnp.float32),
            pltpu.SemaphoreType.DMA,
            pltpu.SemaphoreType.DMA,
            pltpu.SemaphoreType.DMA,
            pltpu.SemaphoreType.DMA,
            pltpu.SemaphoreType.DMA,
            pltpu.SemaphoreType.DMA,
            pltpu.SemaphoreType.DMA,
            pltpu.SemaphoreType.DMA,
            pltpu.SemaphoreType.DMA,
            pltpu.SemaphoreType.DMA,
            pltpu.SemaphoreType.DMA,
        ],
    )
    def emb_kernel(xr_hbm, tok_hbm, pos_hbm, out_hbm,
                   idx_v, oidx_v, pos_v, t0, t1, t2,
                   gs0, gs1, gs2, ps0, ps1, ps2, os0, os1, os2, isem, osem2):
        c = lax.axis_index("c")
        s = lax.axis_index("s")
        step_base = s * SPP + c * N0

        toks = (t0, t1, t2)
        gss = (gs0, gs1, gs2)
        pss = (ps0, ps1, ps2)
        oss = (os0, os1, os2)

        @pl.when(c == 0)
        def _():
            pltpu.async_copy(xr_hbm.at[pl.ds(step_base * ROWS, N0 * ROWS)],
                             idx_v, isem)

        @pl.when(c == 1)
        def _():
            pltpu.async_copy(xr_hbm.at[pl.ds(step_base * ROWS, N1 * ROWS)],
                             idx_v.at[pl.ds(0, N1 * ROWS)], isem)

        lane = lax.iota(jnp.int32, LANES)
        pat0 = (lane & 3) * S + lax.shift_right_logical(lane, 2)
        pat1 = pat0 + (LANES // B)
        nst = jnp.where(c == 0, N0, N1)

        def fill_oidx(t, carry):
            sbase = (step_base + t) * SCHUNK
            oidx_v[t, pl.ds(0, LANES)] = pat0 + sbase
            oidx_v[t, pl.ds(LANES, LANES)] = pat1 + sbase
            return carry

        lax.fori_loop(0, nst, fill_oidx, 0)

        @pl.when(c == 0)
        def _():
            pltpu.make_async_copy(
                xr_hbm.at[pl.ds(0, N0 * ROWS)], idx_v, isem).wait()

        @pl.when(c == 1)
        def _():
            pltpu.make_async_copy(
                xr_hbm.at[pl.ds(0, N1 * ROWS)],
                idx_v.at[pl.ds(0, N1 * ROWS)], isem).wait()

        def start_step(t):
            p = t % NBUF
            pltpu.async_copy(tok_hbm.at[idx_v.at[pl.ds(t * ROWS, ROWS)]],
                             toks[p], gss[p])
            pltpu.async_copy(
                pos_hbm.at[pl.ds((step_base + t) * SCHUNK, SCHUNK)],
                pos_v.at[p], pss[p])

        def process_step(t):
            p = t % NBUF
            pltpu.make_async_copy(
                tok_hbm.at[idx_v.at[pl.ds(t * ROWS, ROWS)]],
                toks[p], gss[p]).wait()
            pltpu.make_async_copy(
                pos_hbm.at[pl.ds(0, SCHUNK)], pos_v.at[p], pss[p]).wait()

            @plsc.parallel_loop(0, SCHUNK, unroll=2)
            def add_pos(r):
                for cc in range(D_VECS):
                    sl = pl.ds(cc * LANES, LANES)
                    pvec = pos_v[p, r, sl]
                    for bb in range(B):
                        plsc.addupdate(toks[p].at[r * B + bb, sl], pvec)

            pltpu.async_copy(toks[p], out_hbm.at[oidx_v.at[t]], oss[p])

        def wait_out(p):
            pltpu.make_async_copy(toks[p], out_hbm.at[oidx_v.at[0]],
                                  oss[p]).wait()

        def guarded(t, fn):
            if t < N1:
                fn()
            else:
                @pl.when(c == 0)
                def _():
                    fn()

        guarded(0, lambda: start_step(0))
        guarded(1, lambda: start_step(1))
        for t in range(N0):
            guarded(t, lambda t=t: process_step(t))
            if t + 2 < N0:
                if t >= 1:
                    guarded(t + 2, lambda t=t: wait_out((t + 2) % NBUF))
                guarded(t + 2, lambda t=t: start_step(t + 2))

        @pl.when(c == 0)
        def _():
            for t in range(N0 - NBUF, N0):
                wait_out(t % NBUF)

        @pl.when(c == 1)
        def _():
            for t in range(max(N1 - NBUF, 0), N1):
                wait_out(t % NBUF)

    return emb_kernel


_emb_kernel = _make_kernel()


def kernel(x, token_emb, pos_emb):
    xr = (x.astype(jnp.int32)
           .reshape(B, GSTEPS, SCHUNK)
           .transpose(1, 2, 0)
           .reshape(GSTEPS, ROWS))
    out = _emb_kernel(xr.reshape(-1), token_emb, pos_emb)
    return out.reshape(B, S, D)


# speedup vs baseline: 1.0159x; 1.0159x over previous
import functools

import numpy as np

import jax
import jax.numpy as jnp
from jax import lax
from jax.experimental import pallas as pl
from jax.experimental.pallas import tpu as pltpu
from jax.experimental.pallas import tpu_sc as plsc

B = 4
S = 2048
D = 768
LANES = 16
D_VECS = D // LANES

NUM_CORES = 2
NUM_SUBCORES = 16
SCHUNK = 8
ROWS = SCHUNK * B
GSTEPS = (B * S) // ROWS
SPP = GSTEPS // NUM_SUBCORES
N0 = 11
N1 = SPP - N0
NBUF = 3


def _make_kernel():
    mesh = plsc.VectorSubcoreMesh(core_axis_name="c", subcore_axis_name="s")

    @functools.partial(
        pl.kernel,
        mesh=mesh,
        out_type=jax.ShapeDtypeStruct((B * S, D), jnp.float32),
        scratch_types=[
            pltpu.VMEM((N0 * ROWS,), jnp.int32),
            pltpu.VMEM((N0, ROWS), jnp.int32),
            pltpu.VMEM((NBUF, SCHUNK, D), jnp.float32),
            pltpu.VMEM((ROWS, D), jnp.float32),
            pltpu.VMEM((ROWS, D), jnp.float32),
            pltpu.VMEM((ROWS, D), jnp.float32),
            pltpu.SemaphoreType.DMA,
            pltpu.SemaphoreType.DMA,
            pltpu.SemaphoreType.DMA,
            pltpu.SemaphoreType.DMA,
            pltpu.SemaphoreType.DMA,
            pltpu.SemaphoreType.DMA,
            pltpu.SemaphoreType.DMA,
            pltpu.SemaphoreType.DMA,
            pltpu.SemaphoreType.DMA,
            pltpu.SemaphoreType.DMA,
            pltpu.SemaphoreType.DMA,
        ],
    )
    def emb_kernel(xr_hbm, tok_hbm, pos_hbm, out_hbm,
                   idx_v, oidx_v, pos_v, t0, t1, t2,
                   gs0, gs1, gs2, ps0, ps1, ps2, os0, os1, os2, isem, osem2):
        c = lax.axis_index("c")
        s = lax.axis_index("s")
        step_base = s * SPP + (1 - c) * N0

        toks = (t0, t1, t2)
        gss = (gs0, gs1, gs2)
        pss = (ps0, ps1, ps2)
        oss = (os0, os1, os2)

        @pl.when(c == 1)
        def _():
            pltpu.async_copy(xr_hbm.at[pl.ds(step_base * ROWS, N0 * ROWS)],
                             idx_v, isem)

        @pl.when(c == 0)
        def _():
            pltpu.async_copy(xr_hbm.at[pl.ds(step_base * ROWS, N1 * ROWS)],
                             idx_v.at[pl.ds(0, N1 * ROWS)], isem)

        lane = lax.iota(jnp.int32, LANES)
        pat0 = (lane & 3) * S + lax.shift_right_logical(lane, 2)
        pat1 = pat0 + (LANES // B)
        nst = jnp.where(c == 1, N0, N1)

        def fill_oidx(t, carry):
            sbase = (step_base + t) * SCHUNK
            oidx_v[t, pl.ds(0, LANES)] = pat0 + sbase
            oidx_v[t, pl.ds(LANES, LANES)] = pat1 + sbase
            return carry

        lax.fori_loop(0, nst, fill_oidx, 0)

        @pl.when(c == 1)
        def _():
            pltpu.make_async_copy(
                xr_hbm.at[pl.ds(0, N0 * ROWS)], idx_v, isem).wait()

        @pl.when(c == 0)
        def _():
            pltpu.make_async_copy(
                xr_hbm.at[pl.ds(0, N1 * ROWS)],
                idx_v.at[pl.ds(0, N1 * ROWS)], isem).wait()

        def start_step(t):
            p = t % NBUF
            pltpu.async_copy(tok_hbm.at[idx_v.at[pl.ds(t * ROWS, ROWS)]],
                             toks[p], gss[p])
            pltpu.async_copy(
                pos_hbm.at[pl.ds((step_base + t) * SCHUNK, SCHUNK)],
                pos_v.at[p], pss[p])

        def process_step(t):
            p = t % NBUF
            pltpu.make_async_copy(
                tok_hbm.at[idx_v.at[pl.ds(t * ROWS, ROWS)]],
                toks[p], gss[p]).wait()
            pltpu.make_async_copy(
                pos_hbm.at[pl.ds(0, SCHUNK)], pos_v.at[p], pss[p]).wait()

            @plsc.parallel_loop(0, SCHUNK, unroll=2)
            def add_pos(r):
                for cc in range(D_VECS):
                    sl = pl.ds(cc * LANES, LANES)
                    pvec = pos_v[p, r, sl]
                    for bb in range(B):
                        plsc.addupdate(toks[p].at[r * B + bb, sl], pvec)

            pltpu.async_copy(toks[p], out_hbm.at[oidx_v.at[t]], oss[p])

        def wait_out(p):
            pltpu.make_async_copy(toks[p], out_hbm.at[oidx_v.at[0]],
                                  oss[p]).wait()

        def guarded(t, fn):
            if t < N1:
                fn()
            else:
                @pl.when(c == 1)
                def _():
                    fn()

        guarded(0, lambda: start_step(0))
        guarded(1, lambda: start_step(1))
        for t in range(N0):
            guarded(t, lambda t=t: process_step(t))
            if t + 2 < N0:
                if t >= 1:
                    guarded(t + 2, lambda t=t: wait_out((t + 2) % NBUF))
                guarded(t + 2, lambda t=t: start_step(t + 2))

        @pl.when(c == 1)
        def _():
            for t in range(N0 - NBUF, N0):
                wait_out(t % NBUF)

        @pl.when(c == 0)
        def _():
            for t in range(max(N1 - NBUF, 0), N1):
                wait_out(t % NBUF)

    return emb_kernel


_emb_kernel = _make_kernel()


def kernel(x, token_emb, pos_emb):
    xr = (x.astype(jnp.int32)
           .reshape(B, GSTEPS, SCHUNK)
           .transpose(1, 2, 0)
           .reshape(GSTEPS, ROWS))
    out = _emb_kernel(xr.reshape(-1), token_emb, pos_emb)
    return out.reshape(B, S, D)


# speedup vs baseline: 1.1310x; 1.1133x over previous
import functools

import numpy as np

import jax
import jax.numpy as jnp
from jax import lax
from jax.experimental import pallas as pl
from jax.experimental.pallas import tpu as pltpu
from jax.experimental.pallas import tpu_sc as plsc

B = 4
S = 2048
D = 768
LANES = 16
D_VECS = D // LANES

NUM_CORES = 2
NUM_SUBCORES = 16
NW = NUM_CORES * NUM_SUBCORES
S_PER_W = S // NW
SCHUNK = 8
ROWS = SCHUNK * B
NSTEP = S_PER_W // SCHUNK
NBUF = 3
RPW = B * S_PER_W


def _make_kernel():
    mesh = plsc.VectorSubcoreMesh(core_axis_name="c", subcore_axis_name="s")

    @functools.partial(
        pl.kernel,
        mesh=mesh,
        out_type=jax.ShapeDtypeStruct((B * S, D), jnp.float32),
        scratch_types=[
            pltpu.VMEM((RPW,), jnp.int32),
            pltpu.VMEM((NSTEP, ROWS), jnp.int32),
            pltpu.VMEM((S_PER_W, D), jnp.float32),
            pltpu.VMEM((ROWS, D), jnp.float32),
            pltpu.VMEM((ROWS, D), jnp.float32),
            pltpu.VMEM((ROWS, D), jnp.float32),
            pltpu.SemaphoreType.DMA,
            pltpu.SemaphoreType.DMA,
            pltpu.SemaphoreType.DMA,
            pltpu.SemaphoreType.DMA,
            pltpu.SemaphoreType.DMA,
            pltpu.SemaphoreType.DMA,
            pltpu.SemaphoreType.DMA,
            pltpu.SemaphoreType.DMA,
            pltpu.SemaphoreType.DMA,
        ],
    )
    def emb_kernel(xr_hbm, oidx_hbm, tok_hbm, pos_hbm, out_hbm,
                   idx_v, oidx_v, pos_v, t0, t1, t2,
                   gs0, gs1, gs2, os0, os1, os2, psem, isem, osem2):
        wid = lax.axis_index("s") * NUM_CORES + lax.axis_index("c")
        s_base = wid * S_PER_W

        toks = (t0, t1, t2)
        gss = (gs0, gs1, gs2)
        oss = (os0, os1, os2)

        icp = pltpu.async_copy(xr_hbm.at[pl.ds(wid * RPW, RPW)], idx_v, isem)
        ocp = pltpu.async_copy(
            oidx_hbm.at[pl.ds(wid * NSTEP, NSTEP)], oidx_v, osem2)
        pcp = pltpu.async_copy(pos_hbm.at[pl.ds(s_base, S_PER_W)],
                               pos_v, psem)
        icp.wait()
        ocp.wait()

        def start_step(t):
            p = t % NBUF
            pltpu.async_copy(
                tok_hbm.at[idx_v.at[pl.ds(t * ROWS, ROWS)]],
                toks[p], gss[p])

        def process_step(t):
            p = t % NBUF
            pltpu.make_async_copy(
                tok_hbm.at[idx_v.at[pl.ds(t * ROWS, ROWS)]],
                toks[p], gss[p]).wait()
            pbase = t * SCHUNK

            @plsc.parallel_loop(0, SCHUNK, unroll=2)
            def add_pos(r):
                for c in range(D_VECS):
                    sl = pl.ds(c * LANES, LANES)
                    pvec = pos_v[pbase + r, sl]
                    for bb in range(B):
                        plsc.addupdate(toks[p].at[r * B + bb, sl], pvec)
            pltpu.async_copy(toks[p], out_hbm.at[oidx_v.at[t]], oss[p])

        def wait_out(p, t):
            pltpu.make_async_copy(toks[p], out_hbm.at[oidx_v.at[t]],
                                  oss[p]).wait()

        start_step(0)
        start_step(1)
        pcp.wait()
        for t in range(NSTEP):
            process_step(t)
            if t + 2 < NSTEP:
                if t >= 1:
                    wait_out((t + 2) % NBUF, t - 1)
                start_step(t + 2)
        for t in range(NSTEP - NBUF, NSTEP):
            wait_out(t % NBUF, t)

    return emb_kernel


_emb_kernel = _make_kernel()


_OIDX = jnp.asarray(
    (np.arange(B, dtype=np.int32)[None, :] * S
     + np.arange(S, dtype=np.int32)[:, None]).reshape(NW * NSTEP, ROWS))


def kernel(x, token_emb, pos_emb):
    xr = (x.astype(jnp.int32)
           .reshape(B, NW, S_PER_W)
           .transpose(1, 2, 0)
           .reshape(-1))
    oidx = _OIDX
    out = _emb_kernel(xr, oidx, token_emb, pos_emb)
    return out.reshape(B, S, D)
